# baseline (speedup 1.0000x reference)
import functools
import jax
import jax.numpy as jnp
from jax import lax
from jax.experimental import pallas as pl
from jax.experimental.pallas import tpu as pltpu, tpu_sc as plsc

BATCH = 16384
NUM_FIELDS = 26
EMBED_DIM = 16
NROWS = 2600000
NC = 2
NS = 16
NW = NC * NS
BPW = BATCH // NW
LANES = 16
JBLKS = BPW // LANES

NTC = NROWS // 128
TAILBASE = NTC * 128
TAILROWS = NROWS - TAILBASE
MAIN_PAIRS = 317
TAIL0 = NW * 2 * MAIN_PAIRS


def _sc_detile(t_t):
    mesh = plsc.VectorSubcoreMesh(core_axis_name="c", subcore_axis_name="s")

    @functools.partial(
        pl.kernel,
        out_type=jax.ShapeDtypeStruct((NROWS // 8, 128), jnp.float32),
        mesh=mesh,
        scratch_types=[
            pltpu.VMEM((EMBED_DIM, 128), jnp.float32),
            pltpu.VMEM((EMBED_DIM, 128), jnp.float32),
            pltpu.VMEM((EMBED_DIM, 128), jnp.float32),
            pltpu.VMEM((EMBED_DIM, 128), jnp.float32),
            pltpu.SemaphoreType.DMA,
            pltpu.SemaphoreType.DMA,
            pltpu.SemaphoreType.DMA,
            pltpu.SemaphoreType.DMA,
        ],
        compiler_params=pltpu.CompilerParams(
            use_tc_tiling_on_sc=True, needs_layout_passes=False
        ),
    )
    def k(t_hbm, out_hbm, in_a, in_b, ob_a, ob_b, isem_a, isem_b, osem_a, osem_b):
        wid = lax.axis_index("s") * NC + lax.axis_index("c")
        ins = (in_a, in_b)
        obs = (ob_a, ob_b)
        isems = (isem_a, isem_b)
        osems = (osem_a, osem_b)
        iota = lax.iota(jnp.int32, LANES)

        def col_of(t, s):
            return wid + NW * (2 * t + s)

        def in_start(c, s):
            return pltpu.async_copy(
                t_hbm.at[:, pl.ds(c * 128, 128)], ins[s], isems[s]
            )

        def out_start(c, s):
            return pltpu.async_copy(
                obs[s], out_hbm.at[pl.ds(c * 16, 16)], osems[s]
            )

        def transform(s):
            for l in range(128):
                v = plsc.load_gather(
                    ins[s], [iota, jnp.full((LANES,), l, dtype=jnp.int32)]
                )
                obs[s][l // 8, pl.ds((l % 8) * LANES, LANES)] = v

        in_start(col_of(0, 0), 0)
        in_start(col_of(0, 1), 1)

        def body(t, _):
            for s in (0, 1):
                c = col_of(t, s)
                pltpu.make_async_copy(
                    t_hbm.at[:, pl.ds(c * 128, 128)], ins[s], isems[s]
                ).wait()

                @pl.when(t > 0)
                def _():
                    cp = col_of(t - 1, s)
                    pltpu.make_async_copy(
                        obs[s], out_hbm.at[pl.ds(cp * 16, 16)], osems[s]
                    ).wait()

                transform(s)
                out_start(c, s)

                @pl.when(t < MAIN_PAIRS - 1)
                def _():
                    in_start(col_of(t + 1, s), s)

            return 0

        lax.fori_loop(0, MAIN_PAIRS, body, 0)
        for s in (0, 1):
            pltpu.make_async_copy(
                obs[s],
                out_hbm.at[pl.ds(col_of(MAIN_PAIRS - 1, s) * 16, 16)],
                osems[s],
            ).wait()

        @pl.when(wid < NTC - TAIL0)
        def _():
            c = TAIL0 + wid
            pltpu.sync_copy(t_hbm.at[:, pl.ds(c * 128, 128)], in_a)
            transform(0)
            pltpu.sync_copy(ob_a, out_hbm.at[pl.ds(c * 16, 16)])


    return k(t_t)


def _sc_gather(x_t, table16, off_b, tail):
    mesh = plsc.VectorSubcoreMesh(core_axis_name="c", subcore_axis_name="s")

    @functools.partial(
        pl.kernel,
        out_type=jax.ShapeDtypeStruct((NUM_FIELDS, EMBED_DIM, BATCH), jnp.float32),
        mesh=mesh,
        scratch_types=[
            pltpu.VMEM((BPW,), jnp.int32),
            pltpu.VMEM((BPW,), jnp.int32),
            pltpu.VMEM((NUM_FIELDS, EMBED_DIM), jnp.int32),
            pltpu.VMEM((BPW, EMBED_DIM), jnp.float32),
            pltpu.VMEM((BPW, EMBED_DIM), jnp.float32),
            pltpu.VMEM((EMBED_DIM, BPW), jnp.float32),
            pltpu.VMEM((EMBED_DIM, BPW), jnp.float32),
            pltpu.VMEM((TAILROWS, EMBED_DIM), jnp.float32),
            pltpu.SemaphoreType.DMA,
            pltpu.SemaphoreType.DMA,
            pltpu.SemaphoreType.DMA,
            pltpu.SemaphoreType.DMA,
        ],
        compiler_params=pltpu.CompilerParams(
            use_tc_tiling_on_sc=False, needs_layout_passes=False
        ),
    )
    def k(x_hbm, table_hbm, off_hbm, tail_hbm, out_hbm,
          idx_a, idx_b, off_v, rows_a, rows_b, tr_a, tr_b, tail_v,
          gsem_a, gsem_b, wsem_a, wsem_b):
        wid = lax.axis_index("s") * NC + lax.axis_index("c")
        b0 = wid * BPW

        idxs = (idx_a, idx_b)
        rows = (rows_a, rows_b)
        trs = (tr_a, tr_b)
        gsems = (gsem_a, gsem_b)
        wsems = (wsem_a, wsem_b)

        pltpu.sync_copy(off_hbm, off_v)
        pltpu.sync_copy(tail_hbm, tail_v)
        iota = lax.iota(jnp.int32, LANES)

        def load_add(f, slot):
            pltpu.sync_copy(x_hbm.at[f, pl.ds(b0, BPW)], idxs[slot])
            off_vec = off_v[f, :]

            def body(i, _):
                s = pl.ds(i * LANES, LANES)
                idxs[slot][s] = idxs[slot][s] + off_vec
                return 0

            lax.fori_loop(0, BPW // LANES, body, 0, unroll=4)

        def gather_start(slot):
            return pltpu.async_copy(
                table_hbm.at[idxs[slot]], rows[slot], gsems[slot]
            )

        def transpose(slot, patch_tail=False):
            def body(jb, _):
                row_ids = iota + jb * LANES
                if patch_tail:
                    idxv = idxs[slot][pl.ds(jb * LANES, LANES)]
                    in_tail = idxv >= TAILBASE
                    lidx = jnp.maximum(idxv - TAILBASE, 0)
                for e in range(EMBED_DIM):
                    col_ids = jnp.full((LANES,), e, dtype=jnp.int32)
                    v = plsc.load_gather(rows[slot], [row_ids, col_ids])
                    if patch_tail:
                        vt = plsc.load_gather(tail_v, [lidx, col_ids])
                        v = jnp.where(in_tail, vt, v)
                    trs[slot][e, pl.ds(jb * LANES, LANES)] = v
                return 0

            lax.fori_loop(0, JBLKS, body, 0)

        def write_start(f, slot):
            return pltpu.async_copy(
                trs[slot], out_hbm.at[f, :, pl.ds(b0, BPW)], wsems[slot]
            )

        load_add(0, 0)
        g_pending = gather_start(0)
        w_pending = [None, None]

        for f in range(1, NUM_FIELDS + 1):
            slot = f % 2
            prev = 1 - slot
            if f < NUM_FIELDS:
                load_add(f, slot)
                g_next = gather_start(slot)
            g_pending.wait()
            if w_pending[prev] is not None:
                w_pending[prev].wait()
            transpose(prev, patch_tail=(f - 1 == NUM_FIELDS - 1))
            w_pending[prev] = write_start(f - 1, prev)
            if f < NUM_FIELDS:
                g_pending = g_next

        for d in w_pending:
            if d is not None:
                d.wait()

    return k(x_t, table16, off_b, tail)


@jax.jit
def kernel(x, table, offset):
    x_t = x.astype(jnp.int32).T
    off_b = jnp.tile(offset.astype(jnp.int32)[:, None], (1, EMBED_DIM))
    tail = table[TAILBASE:, :]
    dense = _sc_detile(table.T)
    table16 = dense.reshape(NROWS, EMBED_DIM)
    out_t = _sc_gather(x_t, table16, off_b, tail)
    return jnp.transpose(out_t, (2, 0, 1))


# speedup vs baseline: 1.8720x; 1.8720x over previous
import functools
import jax
import jax.numpy as jnp
from jax import lax
from jax.experimental import pallas as pl
from jax.experimental.pallas import tpu as pltpu, tpu_sc as plsc

BATCH = 16384
NUM_FIELDS = 26
EMBED_DIM = 16
NROWS = 2600000
NC = 2
NS = 16
NW = NC * NS
BPW = BATCH // NW
LANES = 16
JBLKS = BPW // LANES

NTC = NROWS // 128
TAILBASE = NTC * 128
TAILROWS = NROWS - TAILBASE
MAIN_PAIRS = 317
TAIL0 = NW * 2 * MAIN_PAIRS


def _sc_detile(t_t):
    mesh = plsc.VectorSubcoreMesh(core_axis_name="c", subcore_axis_name="s")

    @functools.partial(
        pl.kernel,
        out_type=jax.ShapeDtypeStruct((NROWS // 8, 128), jnp.float32),
        mesh=mesh,
        scratch_types=[
            pltpu.VMEM((EMBED_DIM, 128), jnp.float32),
            pltpu.VMEM((EMBED_DIM, 128), jnp.float32),
            pltpu.VMEM((EMBED_DIM, 128), jnp.float32),
            pltpu.VMEM((EMBED_DIM, 128), jnp.float32),
            pltpu.SemaphoreType.DMA,
            pltpu.SemaphoreType.DMA,
            pltpu.SemaphoreType.DMA,
            pltpu.SemaphoreType.DMA,
        ],
        compiler_params=pltpu.CompilerParams(
            use_tc_tiling_on_sc=True, needs_layout_passes=False
        ),
    )
    def k(t_hbm, out_hbm, in_a, in_b, ob_a, ob_b, isem_a, isem_b, osem_a, osem_b):
        wid = lax.axis_index("s") * NC + lax.axis_index("c")
        ins = (in_a, in_b)
        obs = (ob_a, ob_b)
        isems = (isem_a, isem_b)
        osems = (osem_a, osem_b)
        iota = lax.iota(jnp.int32, LANES)

        def col_of(t, s):
            return wid + NW * (2 * t + s)

        def in_start(c, s):
            return pltpu.async_copy(
                t_hbm.at[:, pl.ds(c * 128, 128)], ins[s], isems[s]
            )

        def out_start(c, s):
            return pltpu.async_copy(
                obs[s], out_hbm.at[pl.ds(c * 16, 16)], osems[s]
            )

        def transform(s):
            for l0 in range(0, 128, 8):
                vs = [
                    plsc.load_gather(
                        ins[s],
                        [iota, jnp.full((LANES,), l0 + i, dtype=jnp.int32)],
                    )
                    for i in range(8)
                ]
                for i in range(8):
                    l = l0 + i
                    obs[s][l // 8, pl.ds((l % 8) * LANES, LANES)] = vs[i]

        in_start(col_of(0, 0), 0)
        in_start(col_of(0, 1), 1)

        def body(t, _):
            for s in (0, 1):
                c = col_of(t, s)
                pltpu.make_async_copy(
                    t_hbm.at[:, pl.ds(c * 128, 128)], ins[s], isems[s]
                ).wait()

                @pl.when(t > 0)
                def _():
                    cp = col_of(t - 1, s)
                    pltpu.make_async_copy(
                        obs[s], out_hbm.at[pl.ds(cp * 16, 16)], osems[s]
                    ).wait()

                transform(s)
                out_start(c, s)

                @pl.when(t < MAIN_PAIRS - 1)
                def _():
                    in_start(col_of(t + 1, s), s)

            return 0

        lax.fori_loop(0, MAIN_PAIRS, body, 0)
        for s in (0, 1):
            pltpu.make_async_copy(
                obs[s],
                out_hbm.at[pl.ds(col_of(MAIN_PAIRS - 1, s) * 16, 16)],
                osems[s],
            ).wait()

        @pl.when(wid < NTC - TAIL0)
        def _():
            c = TAIL0 + wid
            pltpu.sync_copy(t_hbm.at[:, pl.ds(c * 128, 128)], in_a)
            transform(0)
            pltpu.sync_copy(ob_a, out_hbm.at[pl.ds(c * 16, 16)])


    return k(t_t)


def _sc_gather(x_t, table16, off_b, tail):
    mesh = plsc.VectorSubcoreMesh(core_axis_name="c", subcore_axis_name="s")

    @functools.partial(
        pl.kernel,
        out_type=jax.ShapeDtypeStruct((NUM_FIELDS, EMBED_DIM, BATCH), jnp.float32),
        mesh=mesh,
        scratch_types=[
            pltpu.VMEM((BPW,), jnp.int32),
            pltpu.VMEM((BPW,), jnp.int32),
            pltpu.VMEM((NUM_FIELDS, EMBED_DIM), jnp.int32),
            pltpu.VMEM((BPW, EMBED_DIM), jnp.float32),
            pltpu.VMEM((BPW, EMBED_DIM), jnp.float32),
            pltpu.VMEM((EMBED_DIM, BPW), jnp.float32),
            pltpu.VMEM((EMBED_DIM, BPW), jnp.float32),
            pltpu.VMEM((TAILROWS, EMBED_DIM), jnp.float32),
            pltpu.SemaphoreType.DMA,
            pltpu.SemaphoreType.DMA,
            pltpu.SemaphoreType.DMA,
            pltpu.SemaphoreType.DMA,
        ],
        compiler_params=pltpu.CompilerParams(
            use_tc_tiling_on_sc=False, needs_layout_passes=False
        ),
    )
    def k(x_hbm, table_hbm, off_hbm, tail_hbm, out_hbm,
          idx_a, idx_b, off_v, rows_a, rows_b, tr_a, tr_b, tail_v,
          gsem_a, gsem_b, wsem_a, wsem_b):
        wid = lax.axis_index("s") * NC + lax.axis_index("c")
        b0 = wid * BPW

        idxs = (idx_a, idx_b)
        rows = (rows_a, rows_b)
        trs = (tr_a, tr_b)
        gsems = (gsem_a, gsem_b)
        wsems = (wsem_a, wsem_b)

        pltpu.sync_copy(off_hbm, off_v)
        pltpu.sync_copy(tail_hbm, tail_v)
        iota = lax.iota(jnp.int32, LANES)

        def load_add(f, slot):
            pltpu.sync_copy(x_hbm.at[f, pl.ds(b0, BPW)], idxs[slot])
            off_vec = off_v[f, :]

            def body(i, _):
                s = pl.ds(i * LANES, LANES)
                idxs[slot][s] = idxs[slot][s] + off_vec
                return 0

            lax.fori_loop(0, BPW // LANES, body, 0, unroll=4)

        def gather_start(slot):
            return pltpu.async_copy(
                table_hbm.at[idxs[slot]], rows[slot], gsems[slot]
            )

        def transpose(slot, patch_tail=False):
            def body(jb, _):
                row_ids = iota + jb * LANES
                if patch_tail:
                    idxv = idxs[slot][pl.ds(jb * LANES, LANES)]
                    in_tail = idxv >= TAILBASE
                    lidx = jnp.maximum(idxv - TAILBASE, 0)
                vs = []
                for e in range(EMBED_DIM):
                    col_ids = jnp.full((LANES,), e, dtype=jnp.int32)
                    v = plsc.load_gather(rows[slot], [row_ids, col_ids])
                    if patch_tail:
                        vt = plsc.load_gather(tail_v, [lidx, col_ids])
                        v = jnp.where(in_tail, vt, v)
                    vs.append(v)
                for e in range(EMBED_DIM):
                    trs[slot][e, pl.ds(jb * LANES, LANES)] = vs[e]
                return 0

            lax.fori_loop(0, JBLKS, body, 0)

        def write_start(f, slot):
            return pltpu.async_copy(
                trs[slot], out_hbm.at[f, :, pl.ds(b0, BPW)], wsems[slot]
            )

        load_add(0, 0)
        g_pending = gather_start(0)
        w_pending = [None, None]

        for f in range(1, NUM_FIELDS + 1):
            slot = f % 2
            prev = 1 - slot
            if f < NUM_FIELDS:
                load_add(f, slot)
                g_next = gather_start(slot)
            g_pending.wait()
            if w_pending[prev] is not None:
                w_pending[prev].wait()
            transpose(prev, patch_tail=(f - 1 == NUM_FIELDS - 1))
            w_pending[prev] = write_start(f - 1, prev)
            if f < NUM_FIELDS:
                g_pending = g_next

        for d in w_pending:
            if d is not None:
                d.wait()

    return k(x_t, table16, off_b, tail)


@jax.jit
def kernel(x, table, offset):
    x_t = x.astype(jnp.int32).T
    off_b = jnp.tile(offset.astype(jnp.int32)[:, None], (1, EMBED_DIM))
    tail = table[TAILBASE:, :]
    dense = _sc_detile(table.T)
    table16 = dense.reshape(NROWS, EMBED_DIM)
    out_t = _sc_gather(x_t, table16, off_b, tail)
    return jnp.transpose(out_t, (2, 0, 1))


# speedup vs baseline: 2.5210x; 1.3467x over previous
import functools
import jax
import jax.numpy as jnp
from jax import lax
from jax.experimental import pallas as pl
from jax.experimental.pallas import tpu as pltpu, tpu_sc as plsc

BATCH = 16384
NUM_FIELDS = 26
EMBED_DIM = 16
NROWS = 2600000
NC = 2
NS = 16
NW = NC * NS
BPW = BATCH // NW
LANES = 16
JBLKS = BPW // LANES

NTC = NROWS // 128
TAILBASE = NTC * 128
TAILROWS = NROWS - TAILBASE
MAIN_PAIRS = 317
TAIL0 = NW * 2 * MAIN_PAIRS


def _sc_detile(t_t):
    mesh = plsc.VectorSubcoreMesh(core_axis_name="c", subcore_axis_name="s")

    @functools.partial(
        pl.kernel,
        out_type=jax.ShapeDtypeStruct((NROWS // 8, 128), jnp.float32),
        mesh=mesh,
        scratch_types=[
            pltpu.VMEM((EMBED_DIM, 128), jnp.float32),
            pltpu.VMEM((EMBED_DIM, 128), jnp.float32),
            pltpu.VMEM((EMBED_DIM, 128), jnp.float32),
            pltpu.VMEM((EMBED_DIM, 128), jnp.float32),
            pltpu.SemaphoreType.DMA,
            pltpu.SemaphoreType.DMA,
            pltpu.SemaphoreType.DMA,
            pltpu.SemaphoreType.DMA,
        ],
        compiler_params=pltpu.CompilerParams(
            use_tc_tiling_on_sc=True, needs_layout_passes=False
        ),
    )
    def k(t_hbm, out_hbm, in_a, in_b, ob_a, ob_b, isem_a, isem_b, osem_a, osem_b):
        wid = lax.axis_index("s") * NC + lax.axis_index("c")
        ins = (in_a, in_b)
        obs = (ob_a, ob_b)
        isems = (isem_a, isem_b)
        osems = (osem_a, osem_b)
        iota = lax.iota(jnp.int32, LANES)

        def col_of(t, s):
            return wid + NW * (2 * t + s)

        def in_start(c, s):
            return pltpu.async_copy(
                t_hbm.at[:, pl.ds(c * 128, 128)], ins[s], isems[s]
            )

        def out_start(c, s):
            return pltpu.async_copy(
                obs[s], out_hbm.at[pl.ds(c * 16, 16)], osems[s]
            )

        rowv = [iota // 8 + 2 * lb for lb in range(8)]
        colv = [(iota % 8) * LANES + e for e in range(EMBED_DIM)]

        def transform(s):
            for lb in range(8):
                vs = [ins[s][e, pl.ds(lb * LANES, LANES)] for e in range(EMBED_DIM)]
                for e in range(EMBED_DIM):
                    plsc.store_scatter(obs[s], [rowv[lb], colv[e]], vs[e])

        in_start(col_of(0, 0), 0)
        in_start(col_of(0, 1), 1)

        def body(t, _):
            for s in (0, 1):
                c = col_of(t, s)
                pltpu.make_async_copy(
                    t_hbm.at[:, pl.ds(c * 128, 128)], ins[s], isems[s]
                ).wait()

                @pl.when(t > 0)
                def _():
                    cp = col_of(t - 1, s)
                    pltpu.make_async_copy(
                        obs[s], out_hbm.at[pl.ds(cp * 16, 16)], osems[s]
                    ).wait()

                transform(s)
                out_start(c, s)

                @pl.when(t < MAIN_PAIRS - 1)
                def _():
                    in_start(col_of(t + 1, s), s)

            return 0

        lax.fori_loop(0, MAIN_PAIRS, body, 0)
        for s in (0, 1):
            pltpu.make_async_copy(
                obs[s],
                out_hbm.at[pl.ds(col_of(MAIN_PAIRS - 1, s) * 16, 16)],
                osems[s],
            ).wait()

        @pl.when(wid < NTC - TAIL0)
        def _():
            c = TAIL0 + wid
            pltpu.sync_copy(t_hbm.at[:, pl.ds(c * 128, 128)], in_a)
            transform(0)
            pltpu.sync_copy(ob_a, out_hbm.at[pl.ds(c * 16, 16)])


    return k(t_t)


def _sc_gather(x_t, table16, off_b, tail):
    mesh = plsc.VectorSubcoreMesh(core_axis_name="c", subcore_axis_name="s")

    @functools.partial(
        pl.kernel,
        out_type=jax.ShapeDtypeStruct((NUM_FIELDS, EMBED_DIM, BATCH), jnp.float32),
        mesh=mesh,
        scratch_types=[
            pltpu.VMEM((BPW,), jnp.int32),
            pltpu.VMEM((BPW,), jnp.int32),
            pltpu.VMEM((NUM_FIELDS, EMBED_DIM), jnp.int32),
            pltpu.VMEM((BPW, EMBED_DIM), jnp.float32),
            pltpu.VMEM((BPW, EMBED_DIM), jnp.float32),
            pltpu.VMEM((EMBED_DIM, BPW), jnp.float32),
            pltpu.VMEM((EMBED_DIM, BPW), jnp.float32),
            pltpu.VMEM((TAILROWS, EMBED_DIM), jnp.float32),
            pltpu.SemaphoreType.DMA,
            pltpu.SemaphoreType.DMA,
            pltpu.SemaphoreType.DMA,
            pltpu.SemaphoreType.DMA,
        ],
        compiler_params=pltpu.CompilerParams(
            use_tc_tiling_on_sc=False, needs_layout_passes=False
        ),
    )
    def k(x_hbm, table_hbm, off_hbm, tail_hbm, out_hbm,
          idx_a, idx_b, off_v, rows_a, rows_b, tr_a, tr_b, tail_v,
          gsem_a, gsem_b, wsem_a, wsem_b):
        wid = lax.axis_index("s") * NC + lax.axis_index("c")
        b0 = wid * BPW

        idxs = (idx_a, idx_b)
        rows = (rows_a, rows_b)
        trs = (tr_a, tr_b)
        gsems = (gsem_a, gsem_b)
        wsems = (wsem_a, wsem_b)

        pltpu.sync_copy(off_hbm, off_v)
        pltpu.sync_copy(tail_hbm, tail_v)
        iota = lax.iota(jnp.int32, LANES)

        def load_add(f, slot):
            pltpu.sync_copy(x_hbm.at[f, pl.ds(b0, BPW)], idxs[slot])
            off_vec = off_v[f, :]

            def body(i, _):
                s = pl.ds(i * LANES, LANES)
                idxs[slot][s] = idxs[slot][s] + off_vec
                return 0

            lax.fori_loop(0, BPW // LANES, body, 0, unroll=4)

        def gather_start(slot):
            return pltpu.async_copy(
                table_hbm.at[idxs[slot]], rows[slot], gsems[slot]
            )

        def transpose(slot, patch_tail=False):
            def body(jb, _):
                row_ids = iota + jb * LANES
                if patch_tail:
                    idxv = idxs[slot][pl.ds(jb * LANES, LANES)]
                    in_tail = idxv >= TAILBASE
                    lidx = jnp.maximum(idxv - TAILBASE, 0)
                vs = []
                for e in range(EMBED_DIM):
                    col_ids = jnp.full((LANES,), e, dtype=jnp.int32)
                    v = plsc.load_gather(rows[slot], [row_ids, col_ids])
                    if patch_tail:
                        vt = plsc.load_gather(tail_v, [lidx, col_ids])
                        v = jnp.where(in_tail, vt, v)
                    vs.append(v)
                for e in range(EMBED_DIM):
                    trs[slot][e, pl.ds(jb * LANES, LANES)] = vs[e]
                return 0

            lax.fori_loop(0, JBLKS, body, 0)

        def write_start(f, slot):
            return pltpu.async_copy(
                trs[slot], out_hbm.at[f, :, pl.ds(b0, BPW)], wsems[slot]
            )

        load_add(0, 0)
        g_pending = gather_start(0)
        w_pending = [None, None]

        for f in range(1, NUM_FIELDS + 1):
            slot = f % 2
            prev = 1 - slot
            if f < NUM_FIELDS:
                load_add(f, slot)
                g_next = gather_start(slot)
            g_pending.wait()
            if w_pending[prev] is not None:
                w_pending[prev].wait()
            transpose(prev, patch_tail=(f - 1 == NUM_FIELDS - 1))
            w_pending[prev] = write_start(f - 1, prev)
            if f < NUM_FIELDS:
                g_pending = g_next

        for d in w_pending:
            if d is not None:
                d.wait()

    return k(x_t, table16, off_b, tail)


@jax.jit
def kernel(x, table, offset):
    x_t = x.astype(jnp.int32).T
    off_b = jnp.tile(offset.astype(jnp.int32)[:, None], (1, EMBED_DIM))
    tail = table[TAILBASE:, :]
    dense = _sc_detile(table.T)
    table16 = dense.reshape(NROWS, EMBED_DIM)
    out_t = _sc_gather(x_t, table16, off_b, tail)
    return jnp.transpose(out_t, (2, 0, 1))


# speedup vs baseline: 2.6922x; 1.0679x over previous
import functools
import jax
import jax.numpy as jnp
from jax import lax
from jax.experimental import pallas as pl
from jax.experimental.pallas import tpu as pltpu, tpu_sc as plsc

BATCH = 16384
NUM_FIELDS = 26
EMBED_DIM = 16
NROWS = 2600000
NC = 2
NS = 16
NW = NC * NS
BPW = BATCH // NW
LANES = 16
JBLKS = BPW // LANES

NTC = NROWS // 128
TAILBASE = NTC * 128
TAILROWS = NROWS - TAILBASE
BLKC = 4
NBLK = NTC // BLKC
MAIN_PAIRS = 79
TAIL0 = NW * 2 * MAIN_PAIRS


def _sc_detile(t_t):
    mesh = plsc.VectorSubcoreMesh(core_axis_name="c", subcore_axis_name="s")

    @functools.partial(
        pl.kernel,
        out_type=jax.ShapeDtypeStruct((NROWS // 8, 128), jnp.float32),
        mesh=mesh,
        scratch_types=[
            pltpu.VMEM((EMBED_DIM, BLKC * 128), jnp.float32),
            pltpu.VMEM((EMBED_DIM, BLKC * 128), jnp.float32),
            pltpu.VMEM((BLKC * 16, 128), jnp.float32),
            pltpu.VMEM((BLKC * 16, 128), jnp.float32),
            pltpu.SemaphoreType.DMA,
            pltpu.SemaphoreType.DMA,
            pltpu.SemaphoreType.DMA,
            pltpu.SemaphoreType.DMA,
        ],
        compiler_params=pltpu.CompilerParams(
            use_tc_tiling_on_sc=True, needs_layout_passes=False
        ),
    )
    def k(t_hbm, out_hbm, in_a, in_b, ob_a, ob_b, isem_a, isem_b, osem_a, osem_b):
        wid = lax.axis_index("s") * NC + lax.axis_index("c")
        ins = (in_a, in_b)
        obs = (ob_a, ob_b)
        isems = (isem_a, isem_b)
        osems = (osem_a, osem_b)
        iota = lax.iota(jnp.int32, LANES)

        def blk_of(t, s):
            return wid + NW * (2 * t + s)

        def in_start(b, s):
            return pltpu.async_copy(
                t_hbm.at[:, pl.ds(b * (BLKC * 128), BLKC * 128)], ins[s], isems[s]
            )

        def out_start(b, s):
            return pltpu.async_copy(
                obs[s], out_hbm.at[pl.ds(b * (BLKC * 16), BLKC * 16)], osems[s]
            )

        rowv = [
            [iota // 8 + 2 * lb + 16 * j for lb in range(8)] for j in range(BLKC)
        ]
        colv = [(iota % 8) * LANES + e for e in range(EMBED_DIM)]

        def transform(s):
            for j in range(BLKC):
                for lb in range(8):
                    vs = [
                        ins[s][e, pl.ds(j * 128 + lb * LANES, LANES)]
                        for e in range(EMBED_DIM)
                    ]
                    for e in range(EMBED_DIM):
                        plsc.store_scatter(obs[s], [rowv[j][lb], colv[e]], vs[e])

        in_start(blk_of(0, 0), 0)
        in_start(blk_of(0, 1), 1)

        def body(t, _):
            for s in (0, 1):
                b = blk_of(t, s)
                pltpu.make_async_copy(
                    t_hbm.at[:, pl.ds(b * (BLKC * 128), BLKC * 128)], ins[s], isems[s]
                ).wait()

                @pl.when(t > 0)
                def _():
                    bp = blk_of(t - 1, s)
                    pltpu.make_async_copy(
                        obs[s],
                        out_hbm.at[pl.ds(bp * (BLKC * 16), BLKC * 16)],
                        osems[s],
                    ).wait()

                transform(s)
                out_start(b, s)

                @pl.when(t < MAIN_PAIRS - 1)
                def _():
                    in_start(blk_of(t + 1, s), s)

            return 0

        lax.fori_loop(0, MAIN_PAIRS, body, 0)
        for s in (0, 1):
            pltpu.make_async_copy(
                obs[s],
                out_hbm.at[pl.ds(blk_of(MAIN_PAIRS - 1, s) * (BLKC * 16), BLKC * 16)],
                osems[s],
            ).wait()

        @pl.when(wid < NBLK - TAIL0)
        def _():
            b = TAIL0 + wid
            pltpu.sync_copy(
                t_hbm.at[:, pl.ds(b * (BLKC * 128), BLKC * 128)], in_a
            )
            transform(0)
            pltpu.sync_copy(
                ob_a, out_hbm.at[pl.ds(b * (BLKC * 16), BLKC * 16)]
            )


    return k(t_t)


def _sc_gather(x_t, table16, off_b, tail):
    mesh = plsc.VectorSubcoreMesh(core_axis_name="c", subcore_axis_name="s")

    @functools.partial(
        pl.kernel,
        out_type=jax.ShapeDtypeStruct((NUM_FIELDS, EMBED_DIM, BATCH), jnp.float32),
        mesh=mesh,
        scratch_types=[
            pltpu.VMEM((BPW,), jnp.int32),
            pltpu.VMEM((BPW,), jnp.int32),
            pltpu.VMEM((NUM_FIELDS, EMBED_DIM), jnp.int32),
            pltpu.VMEM((BPW, EMBED_DIM), jnp.float32),
            pltpu.VMEM((BPW, EMBED_DIM), jnp.float32),
            pltpu.VMEM((EMBED_DIM, BPW), jnp.float32),
            pltpu.VMEM((EMBED_DIM, BPW), jnp.float32),
            pltpu.VMEM((TAILROWS, EMBED_DIM), jnp.float32),
            pltpu.SemaphoreType.DMA,
            pltpu.SemaphoreType.DMA,
            pltpu.SemaphoreType.DMA,
            pltpu.SemaphoreType.DMA,
        ],
        compiler_params=pltpu.CompilerParams(
            use_tc_tiling_on_sc=False, needs_layout_passes=False
        ),
    )
    def k(x_hbm, table_hbm, off_hbm, tail_hbm, out_hbm,
          idx_a, idx_b, off_v, rows_a, rows_b, tr_a, tr_b, tail_v,
          gsem_a, gsem_b, wsem_a, wsem_b):
        wid = lax.axis_index("s") * NC + lax.axis_index("c")
        b0 = wid * BPW

        idxs = (idx_a, idx_b)
        rows = (rows_a, rows_b)
        trs = (tr_a, tr_b)
        gsems = (gsem_a, gsem_b)
        wsems = (wsem_a, wsem_b)

        pltpu.sync_copy(off_hbm, off_v)
        pltpu.sync_copy(tail_hbm, tail_v)
        iota = lax.iota(jnp.int32, LANES)

        def load_add(f, slot):
            pltpu.sync_copy(x_hbm.at[f, pl.ds(b0, BPW)], idxs[slot])
            off_vec = off_v[f, :]

            def body(i, _):
                s = pl.ds(i * LANES, LANES)
                idxs[slot][s] = idxs[slot][s] + off_vec
                return 0

            lax.fori_loop(0, BPW // LANES, body, 0, unroll=4)

        def gather_start(slot):
            return pltpu.async_copy(
                table_hbm.at[idxs[slot]], rows[slot], gsems[slot]
            )

        def transpose(slot, patch_tail=False):
            def body(jb, _):
                row_ids = iota + jb * LANES
                if patch_tail:
                    idxv = idxs[slot][pl.ds(jb * LANES, LANES)]
                    in_tail = idxv >= TAILBASE
                    lidx = jnp.maximum(idxv - TAILBASE, 0)
                vs = []
                for e in range(EMBED_DIM):
                    col_ids = jnp.full((LANES,), e, dtype=jnp.int32)
                    v = plsc.load_gather(rows[slot], [row_ids, col_ids])
                    if patch_tail:
                        vt = plsc.load_gather(tail_v, [lidx, col_ids])
                        v = jnp.where(in_tail, vt, v)
                    vs.append(v)
                for e in range(EMBED_DIM):
                    trs[slot][e, pl.ds(jb * LANES, LANES)] = vs[e]
                return 0

            lax.fori_loop(0, JBLKS, body, 0)

        def write_start(f, slot):
            return pltpu.async_copy(
                trs[slot], out_hbm.at[f, :, pl.ds(b0, BPW)], wsems[slot]
            )

        load_add(0, 0)
        g_pending = gather_start(0)
        w_pending = [None, None]

        for f in range(1, NUM_FIELDS + 1):
            slot = f % 2
            prev = 1 - slot
            if f < NUM_FIELDS:
                load_add(f, slot)
                g_next = gather_start(slot)
            g_pending.wait()
            if w_pending[prev] is not None:
                w_pending[prev].wait()
            transpose(prev, patch_tail=(f - 1 == NUM_FIELDS - 1))
            w_pending[prev] = write_start(f - 1, prev)
            if f < NUM_FIELDS:
                g_pending = g_next

        for d in w_pending:
            if d is not None:
                d.wait()

    return k(x_t, table16, off_b, tail)


@jax.jit
def kernel(x, table, offset):
    x_t = x.astype(jnp.int32).T
    off_b = jnp.tile(offset.astype(jnp.int32)[:, None], (1, EMBED_DIM))
    tail = table[TAILBASE:, :]
    dense = _sc_detile(table.T)
    table16 = dense.reshape(NROWS, EMBED_DIM)
    out_t = _sc_gather(x_t, table16, off_b, tail)
    return jnp.transpose(out_t, (2, 0, 1))
